# baseline (speedup 1.0000x reference)
import functools

import jax
import jax.numpy as jnp
from jax import lax
from jax.experimental import pallas as pl
from jax.experimental.pallas import tpu as pltpu
from jax.experimental.pallas import tpu_sc as plsc

BATCH = 16384
EMB_DIM = 32
LANES = 16

_info = plsc.get_sparse_core_info()
_NC, _NS = _info.num_cores, _info.num_subcores
NW = _NC * _NS
B_PER_W = BATCH // NW
IDX_CHUNK = 128
N_CHUNKS = B_PER_W // IDX_CHUNK


def _mf_kernel(user_hbm, item_hbm, uemb_hbm, iemb_hbm, ubias_hbm, ibias_hbm,
               out_hbm, uidx_v, iidx_v, urows_v, irows_v, ubias_v, ibias_v,
               out_v, sem):
    wid = lax.axis_index("s") * _NC + lax.axis_index("c")
    base = wid * B_PER_W

    pltpu.sync_copy(user_hbm.at[pl.ds(base, B_PER_W)], uidx_v)
    pltpu.sync_copy(item_hbm.at[pl.ds(base, B_PER_W)], iidx_v)

    handles = []
    for j in range(N_CHUNKS):
        sl = pl.ds(j * IDX_CHUNK, IDX_CHUNK)
        handles.append(pltpu.async_copy(
            uemb_hbm.at[uidx_v.at[sl]], urows_v.at[sl], sem))
        handles.append(pltpu.async_copy(
            iemb_hbm.at[iidx_v.at[sl]], irows_v.at[sl], sem))
        handles.append(pltpu.async_copy(
            ubias_hbm.at[uidx_v.at[sl]], ubias_v.at[sl], sem))
        handles.append(pltpu.async_copy(
            ibias_hbm.at[iidx_v.at[sl]], ibias_v.at[sl], sem))
    for h in handles:
        h.wait()

    lane_ids = lax.iota(jnp.int32, LANES)

    def body(g, _):
        rbase = g * LANES
        acc = jnp.zeros((LANES,), jnp.float32)
        for b in range(LANES):
            r = rbase + b
            u_lo = urows_v[r, pl.ds(0, LANES)]
            u_hi = urows_v[r, pl.ds(LANES, LANES)]
            i_lo = irows_v[r, pl.ds(0, LANES)]
            i_hi = irows_v[r, pl.ds(LANES, LANES)]
            s = jnp.sum(u_lo * i_lo + u_hi * i_hi)
            acc = jnp.where(lane_ids == b, s, acc)
        acc += ubias_v[pl.ds(rbase, LANES)] + ibias_v[pl.ds(rbase, LANES)]
        out_v[pl.ds(rbase, LANES)] = acc
        return 0

    lax.fori_loop(0, B_PER_W // LANES, body, 0)

    pltpu.sync_copy(out_v, out_hbm.at[pl.ds(base, B_PER_W)])


@jax.jit
def _mf(user, item, user_embedding, item_embedding, ubias_flat, ibias_flat):
    mesh = plsc.VectorSubcoreMesh(core_axis_name="c", subcore_axis_name="s")
    run = functools.partial(
        pl.kernel,
        mesh=mesh,
        out_type=jax.ShapeDtypeStruct((BATCH,), jnp.float32),
        compiler_params=pltpu.CompilerParams(
            needs_layout_passes=False, use_tc_tiling_on_sc=False),
        scratch_types=[
            pltpu.VMEM((B_PER_W,), jnp.int32),
            pltpu.VMEM((B_PER_W,), jnp.int32),
            pltpu.VMEM((B_PER_W, EMB_DIM), jnp.float32),
            pltpu.VMEM((B_PER_W, EMB_DIM), jnp.float32),
            pltpu.VMEM((B_PER_W,), jnp.float32),
            pltpu.VMEM((B_PER_W,), jnp.float32),
            pltpu.VMEM((B_PER_W,), jnp.float32),
            pltpu.SemaphoreType.DMA,
        ],
    )(_mf_kernel)
    return run(user, item, user_embedding, item_embedding,
               ubias_flat, ibias_flat)


def kernel(user, item, user_embedding, item_embedding, user_bias, item_bias):
    return _mf(user.astype(jnp.int32), item.astype(jnp.int32),
               user_embedding, item_embedding,
               user_bias.reshape(-1), item_bias.reshape(-1))
